# Initial kernel scaffold; baseline (speedup 1.0000x reference)
import jax
import jax.numpy as jnp
from jax.experimental import pallas as pl


def kernel(z, u, x, edge_index, batch, batch_size, W_proj, b_proj, W_ih, W_hh, b_ih, b_hh):
    raise NotImplementedError("write your pallas kernel here")


# speedup vs baseline: 2.3994x; 2.3994x over previous
import functools

import jax
import jax.numpy as jnp
from jax import lax
from jax.experimental import pallas as pl
from jax.experimental.pallas import tpu as pltpu
from jax.experimental.pallas import tpu_sc as plsc

N_NODES = 10000
D_Z = 256
B_GRAPHS = 64
H_OUT = 256
LANES = 16
CH = D_Z // LANES

NC = 2
NS = 16
NW = NC * NS

ROWS = N_NODES // NW
LEFT = N_NODES - NW * ROWS
MAX_ROWS = ROWS + LEFT

_mesh = plsc.VectorSubcoreMesh(core_axis_name="c", subcore_axis_name="s")


@functools.partial(
    pl.kernel,
    out_type=jax.ShapeDtypeStruct((NW, B_GRAPHS, D_Z), jnp.float32),
    mesh=_mesh,
    scratch_types=[
        pltpu.VMEM((MAX_ROWS + 24,), jnp.int32),
        pltpu.VMEM((MAX_ROWS, D_Z), jnp.float32),
        pltpu.VMEM((B_GRAPHS, D_Z), jnp.float32),
        pltpu.SemaphoreType.DMA,
    ],
)
def _segment_sum_sc(z_hbm, batch_hbm, out_hbm, idx_v, rows_v, acc_v, sem):
    cid = lax.axis_index("c")
    sid = lax.axis_index("s")
    wid = cid * NS + sid
    base = wid * ROWS

    rd = pltpu.async_copy(z_hbm.at[pl.ds(base, ROWS)],
                          rows_v.at[pl.ds(0, ROWS)], sem)
    pltpu.sync_copy(batch_hbm.at[pl.ds(base, ROWS)], idx_v.at[pl.ds(0, ROWS)])

    @pl.when(wid == NW - 1)
    def _():
        pltpu.sync_copy(z_hbm.at[pl.ds(NW * ROWS, LEFT)],
                        rows_v.at[pl.ds(ROWS, LEFT)])
        pltpu.sync_copy(batch_hbm.at[pl.ds(NW * ROWS, LEFT)],
                        idx_v.at[pl.ds(ROWS, LEFT)])

    zvec = jnp.zeros((LANES,), jnp.float32)

    def zero_body(i, carry):
        for j in range(CH):
            acc_v[i, pl.ds(j * LANES, LANES)] = zvec
        return carry

    lax.fori_loop(0, B_GRAPHS, zero_body, 0)
    rd.wait()

    n_rows = jnp.where(wid == NW - 1, ROWS + LEFT, ROWS)

    def row_body(i, carry):
        s = idx_v[pl.ds(i, LANES)][0]
        for j in range(CH):
            plsc.addupdate(acc_v.at[s, pl.ds(j * LANES, LANES)],
                           rows_v[i, pl.ds(j * LANES, LANES)])
        return carry

    lax.fori_loop(0, n_rows, row_body, 0)

    pltpu.sync_copy(acc_v, out_hbm.at[wid])


def _fuse_gru_tc(p_ref, u_ref, wp_ref, bp_ref, wih_ref, bih_ref, bhh_ref,
                 out_ref):
    graph_emb = jnp.sum(p_ref[...], axis=0)
    glob = lax.dot_general(u_ref[:], wp_ref[:], (((1,), (1,)), ((), ())),
                           preferred_element_type=jnp.float32)
    glob = jnp.maximum(glob + bp_ref[:], 0.0)
    fused = jnp.concatenate([graph_emb, glob], axis=1)
    gi = lax.dot_general(fused, wih_ref[:], (((1,), (1,)), ((), ())),
                         preferred_element_type=jnp.float32) + bih_ref[:]
    bhh = bhh_ref[:]
    r = jax.nn.sigmoid(gi[:, :H_OUT] + bhh[:, :H_OUT])
    zg = jax.nn.sigmoid(gi[:, H_OUT:2 * H_OUT] + bhh[:, H_OUT:2 * H_OUT])
    n = jnp.tanh(gi[:, 2 * H_OUT:] + r * bhh[:, 2 * H_OUT:])
    out_ref[:] = (1.0 - zg) * n


def kernel(z, u, x, edge_index, batch, batch_size, W_proj, b_proj, W_ih, W_hh,
           b_ih, b_hh):
    del x, edge_index, batch_size, W_hh
    partials = _segment_sum_sc(z, batch.astype(jnp.int32))
    h = pl.pallas_call(
        _fuse_gru_tc,
        out_shape=jax.ShapeDtypeStruct((B_GRAPHS, H_OUT), jnp.float32),
    )(partials, u, W_proj, b_proj.reshape(1, -1), W_ih, b_ih.reshape(1, -1),
      b_hh.reshape(1, -1))
    return (h, h)


# speedup vs baseline: 2.4873x; 1.0366x over previous
import functools

import jax
import jax.numpy as jnp
from jax import lax
from jax.experimental import pallas as pl
from jax.experimental.pallas import tpu as pltpu
from jax.experimental.pallas import tpu_sc as plsc

N_NODES = 10000
D_Z = 256
B_GRAPHS = 64
H_OUT = 256
LANES = 16
CH = D_Z // LANES

NC = 2
NS = 16
NW = NC * NS

G_HI = 20
G_LO = 19
HI_TILES = 17
MAX_ROWS = G_HI * LANES

_mesh = plsc.VectorSubcoreMesh(core_axis_name="c", subcore_axis_name="s")


@functools.partial(
    pl.kernel,
    out_type=jax.ShapeDtypeStruct((NW, B_GRAPHS, D_Z), jnp.float32),
    mesh=_mesh,
    scratch_types=[
        pltpu.VMEM((MAX_ROWS,), jnp.int32),
        pltpu.VMEM((MAX_ROWS, D_Z), jnp.float32),
        pltpu.VMEM((B_GRAPHS, D_Z), jnp.float32),
        pltpu.SemaphoreType.DMA,
    ],
)
def _segment_sum_sc(z_hbm, batch_hbm, out_hbm, idx_v, rows_v, acc_v, sem):
    cid = lax.axis_index("c")
    sid = lax.axis_index("s")
    wid = cid * NS + sid
    base = G_LO * LANES * wid + LANES * jnp.minimum(wid, HI_TILES)

    rd = pltpu.async_copy(z_hbm.at[pl.ds(base, G_LO * LANES)],
                          rows_v.at[pl.ds(0, G_LO * LANES)], sem)
    pltpu.sync_copy(batch_hbm.at[pl.ds(base, G_LO * LANES)],
                    idx_v.at[pl.ds(0, G_LO * LANES)])

    @pl.when(wid < HI_TILES)
    def _():
        pltpu.sync_copy(z_hbm.at[pl.ds(base + G_LO * LANES, LANES)],
                        rows_v.at[pl.ds(G_LO * LANES, LANES)])
        pltpu.sync_copy(batch_hbm.at[pl.ds(base + G_LO * LANES, LANES)],
                        idx_v.at[pl.ds(G_LO * LANES, LANES)])

    zvec = jnp.zeros((LANES,), jnp.float32)

    def zero_body(i, carry):
        for j in range(CH):
            acc_v[i, pl.ds(j * LANES, LANES)] = zvec
        return carry

    lax.fori_loop(0, B_GRAPHS, zero_body, 0)
    rd.wait()

    n_groups = jnp.where(wid < HI_TILES, G_HI, G_LO)

    def group_body(g, carry):
        sv = idx_v[pl.ds(g * LANES, LANES)]
        for k in range(LANES):
            s = sv[k]
            r = g * LANES + k
            for j in range(CH):
                plsc.addupdate(acc_v.at[s, pl.ds(j * LANES, LANES)],
                               rows_v[r, pl.ds(j * LANES, LANES)])
        return carry

    lax.fori_loop(0, n_groups, group_body, 0)

    pltpu.sync_copy(acc_v, out_hbm.at[wid])


def _fuse_gru_tc(p_ref, u_ref, wp_ref, bp_ref, wih_ref, bih_ref, bhh_ref,
                 out_ref):
    graph_emb = jnp.sum(p_ref[...], axis=0)
    glob = lax.dot_general(u_ref[:], wp_ref[:], (((1,), (1,)), ((), ())),
                           preferred_element_type=jnp.float32)
    glob = jnp.maximum(glob + bp_ref[:], 0.0)
    fused = jnp.concatenate([graph_emb, glob], axis=1)
    gi = lax.dot_general(fused, wih_ref[:], (((1,), (1,)), ((), ())),
                         preferred_element_type=jnp.float32) + bih_ref[:]
    bhh = bhh_ref[:]
    r = jax.nn.sigmoid(gi[:, :H_OUT] + bhh[:, :H_OUT])
    zg = jax.nn.sigmoid(gi[:, H_OUT:2 * H_OUT] + bhh[:, H_OUT:2 * H_OUT])
    n = jnp.tanh(gi[:, 2 * H_OUT:] + r * bhh[:, 2 * H_OUT:])
    out_ref[:] = (1.0 - zg) * n


def kernel(z, u, x, edge_index, batch, batch_size, W_proj, b_proj, W_ih, W_hh,
           b_ih, b_hh):
    del x, edge_index, batch_size, W_hh
    partials = _segment_sum_sc(z, batch.astype(jnp.int32))
    h = pl.pallas_call(
        _fuse_gru_tc,
        out_shape=jax.ShapeDtypeStruct((B_GRAPHS, H_OUT), jnp.float32),
    )(partials, u, W_proj, b_proj.reshape(1, -1), W_ih, b_ih.reshape(1, -1),
      b_hh.reshape(1, -1))
    return (h, h)
